# baseline (speedup 1.0000x reference)
import math

import jax
import jax.numpy as jnp
from jax import lax
from jax.experimental import pallas as pl
from jax.experimental.pallas import tpu as pltpu
from jax.experimental.pallas import tpu_sc as plsc

NC = 2
NS = 16
NW = NC * NS
LANES = 16
NBUF = 5
SEQ_BLOCKS = 8
POS_BLOCKS = 4


def _make_body(batch, seq, emb):
    seq_per_w = batch // SEQ_BLOCKS
    pos_per_w = seq // POS_BLOCKS
    n_outer = pos_per_w // NBUF
    scale = math.sqrt(emb)
    nvec = emb // LANES

    def body(idx_t_hbm, table_hbm, pos_hbm, out_hbm, ibuf, rows, pos_v, oidx,
             sg0, sg1, sg2, sg3, sg4, sw0, sw1, sw2, sw3, sw4, sem_i, sem_p):
        sem_g = [sg0, sg1, sg2, sg3, sg4]
        sem_w = [sw0, sw1, sw2, sw3, sw4]
        wid = lax.axis_index("s") * NC + lax.axis_index("c")
        sb = lax.rem(wid, SEQ_BLOCKS)
        pb = wid // SEQ_BLOCKS
        seq0 = sb * seq_per_w
        p0 = pb * pos_per_w

        pltpu.async_copy(pos_hbm.at[pb], pos_v, sem_p).wait()

        def fire_idx(p, slot):
            pltpu.async_copy(idx_t_hbm.at[p0 + p, sb], ibuf.at[slot], sem_i)

        def wait_idx():
            pltpu.make_async_copy(idx_t_hbm.at[0, 0], ibuf.at[0],
                                  sem_i).wait()

        def fire_gather(b):
            pltpu.async_copy(table_hbm.at[ibuf.at[b]], rows.at[b], sem_g[b])

        def wait_gather(b):
            pltpu.make_async_copy(out_hbm.at[pl.ds(0, seq_per_w)],
                                  rows.at[0], sem_g[b]).wait()

        def fire_scatter(b):
            pltpu.async_copy(rows.at[b], out_hbm.at[oidx.at[b]], sem_w[b])

        def wait_scatter(b):
            pltpu.make_async_copy(rows.at[0], out_hbm.at[pl.ds(0, seq_per_w)],
                                  sem_w[b]).wait()

        lane = lax.iota(jnp.int32, LANES) * seq

        def compute(b, p):
            base = (seq0 * seq) + p0 + p
            pv = [pos_v[p, pl.ds(j * LANES, LANES)] for j in range(nvec)]
            for j in range(nvec):
                oidx[b, pl.ds(j * LANES, LANES)] = lane + (
                    base + j * LANES * seq)

            def row_body(r, rc):
                for j in range(nvec):
                    sl = pl.ds(j * LANES, LANES)
                    rows[b, r, sl] = rows[b, r, sl] * scale + pv[j]
                return rc

            lax.fori_loop(0, seq_per_w, row_body, 0)

        fire_idx(0, 0)
        fire_idx(1, 1)
        wait_idx()
        wait_idx()
        fire_gather(0)
        fire_gather(1)
        fire_idx(2, 2)

        def outer(it, c):
            for b in range(NBUF):
                p = it * NBUF + b

                @pl.when(p + 2 < pos_per_w)
                def _():
                    wait_idx()

                    @pl.when(p + 2 >= NBUF)
                    def _():
                        wait_scatter((b + 2) % NBUF)

                    fire_gather((b + 2) % NBUF)

                wait_gather(b)

                @pl.when(p + 3 < pos_per_w)
                def _():
                    fire_idx(p + 3, (b + 3) % NBUF)

                compute(b, p)
                fire_scatter(b)
            return c

        lax.fori_loop(0, n_outer, outer, 0)
        for b in range(NBUF):
            wait_scatter(b)

    return body


def kernel(to_emb, token_table, pos_table):
    batch, seq = to_emb.shape
    emb = token_table.shape[1]
    seq_per_w = batch // SEQ_BLOCKS
    pos_per_w = seq // POS_BLOCKS
    idx_t = to_emb.T.reshape(seq, SEQ_BLOCKS, seq_per_w)
    pos = pos_table[:seq].reshape(POS_BLOCKS, pos_per_w, emb)

    mesh = plsc.VectorSubcoreMesh(core_axis_name="c", subcore_axis_name="s")
    f = pl.kernel(
        _make_body(batch, seq, emb),
        mesh=mesh,
        out_type=jax.ShapeDtypeStruct((batch * seq, emb), jnp.float32),
        scratch_types=[
            pltpu.VMEM((NBUF, seq_per_w), jnp.int32),
            pltpu.VMEM((NBUF, seq_per_w, emb), jnp.float32),
            pltpu.VMEM((pos_per_w, emb), jnp.float32),
            pltpu.VMEM((NBUF, seq_per_w), jnp.int32),
        ] + [pltpu.SemaphoreType.DMA] * 12,
    )
    return f(idx_t, token_table, pos).reshape(batch, seq, emb)


# speedup vs baseline: 1.0069x; 1.0069x over previous
import math

import jax
import jax.numpy as jnp
from jax import lax
from jax.experimental import pallas as pl
from jax.experimental.pallas import tpu as pltpu
from jax.experimental.pallas import tpu_sc as plsc

NC = 2
NS = 16
NW = NC * NS
LANES = 16
NBUF = 5
SEQ_BLOCKS = 8
POS_BLOCKS = 4


def _make_body(batch, seq, emb):
    seq_per_w = batch // SEQ_BLOCKS
    pos_per_w = seq // POS_BLOCKS
    n_outer = pos_per_w // NBUF
    scale = math.sqrt(emb)
    nvec = emb // LANES

    def body(idx_t_hbm, table_hbm, pos_hbm, out_hbm, ibuf, rows, pos_v, oidx,
             sg0, sg1, sg2, sg3, sg4, sw0, sw1, sw2, sw3, sw4, sem_i, sem_p):
        sem_g = [sg0, sg1, sg2, sg3, sg4]
        sem_w = [sw0, sw1, sw2, sw3, sw4]
        wid = lax.axis_index("s") * NC + lax.axis_index("c")
        sb = lax.rem(wid, SEQ_BLOCKS)
        pb = wid // SEQ_BLOCKS
        seq0 = sb * seq_per_w
        p0 = pb * pos_per_w

        pos_cp = pltpu.async_copy(pos_hbm.at[pb], pos_v, sem_p)

        def fire_idx(p, slot):
            pltpu.async_copy(idx_t_hbm.at[p0 + p, sb], ibuf.at[slot], sem_i)

        def wait_idx():
            pltpu.make_async_copy(idx_t_hbm.at[0, 0], ibuf.at[0],
                                  sem_i).wait()

        def fire_gather(b):
            pltpu.async_copy(table_hbm.at[ibuf.at[b]], rows.at[b], sem_g[b])

        def wait_gather(b):
            pltpu.make_async_copy(out_hbm.at[pl.ds(0, seq_per_w)],
                                  rows.at[0], sem_g[b]).wait()

        def fire_scatter(b):
            pltpu.async_copy(rows.at[b], out_hbm.at[oidx.at[b]], sem_w[b])

        def wait_scatter(b):
            pltpu.make_async_copy(rows.at[0], out_hbm.at[pl.ds(0, seq_per_w)],
                                  sem_w[b]).wait()

        lane = lax.iota(jnp.int32, LANES) * seq

        def compute(b, p):
            base = (seq0 * seq) + p0 + p
            pv = [pos_v[p, pl.ds(j * LANES, LANES)] for j in range(nvec)]
            for j in range(nvec):
                oidx[b, pl.ds(j * LANES, LANES)] = lane + (
                    base + j * LANES * seq)

            def row_body(r, rc):
                for j in range(nvec):
                    sl = pl.ds(j * LANES, LANES)
                    rows[b, r, sl] = rows[b, r, sl] * scale + pv[j]
                return rc

            lax.fori_loop(0, seq_per_w, row_body, 0)

        fire_idx(0, 0)
        fire_idx(1, 1)
        fire_idx(2, 2)
        wait_idx()
        wait_idx()
        wait_idx()
        fire_gather(0)
        fire_gather(1)
        fire_gather(2)
        fire_idx(3, 3)
        pos_cp.wait()

        def outer(it, c):
            for b in range(NBUF):
                p = it * NBUF + b

                @pl.when(p + 3 < pos_per_w)
                def _():
                    wait_idx()

                    @pl.when(p + 3 >= NBUF)
                    def _():
                        wait_scatter((b + 3) % NBUF)

                    fire_gather((b + 3) % NBUF)

                wait_gather(b)

                @pl.when(p + 4 < pos_per_w)
                def _():
                    fire_idx(p + 4, (b + 4) % NBUF)

                compute(b, p)
                fire_scatter(b)
            return c

        lax.fori_loop(0, n_outer, outer, 0)
        for b in range(NBUF):
            wait_scatter(b)

    return body


def kernel(to_emb, token_table, pos_table):
    batch, seq = to_emb.shape
    emb = token_table.shape[1]
    seq_per_w = batch // SEQ_BLOCKS
    pos_per_w = seq // POS_BLOCKS
    idx_t = to_emb.T.reshape(seq, SEQ_BLOCKS, seq_per_w)
    pos = pos_table[:seq].reshape(POS_BLOCKS, pos_per_w, emb)

    mesh = plsc.VectorSubcoreMesh(core_axis_name="c", subcore_axis_name="s")
    f = pl.kernel(
        _make_body(batch, seq, emb),
        mesh=mesh,
        out_type=jax.ShapeDtypeStruct((batch * seq, emb), jnp.float32),
        scratch_types=[
            pltpu.VMEM((NBUF, seq_per_w), jnp.int32),
            pltpu.VMEM((NBUF, seq_per_w, emb), jnp.float32),
            pltpu.VMEM((pos_per_w, emb), jnp.float32),
            pltpu.VMEM((NBUF, seq_per_w), jnp.int32),
        ] + [pltpu.SemaphoreType.DMA] * 12,
    )
    return f(idx_t, token_table, pos).reshape(batch, seq, emb)


# speedup vs baseline: 1.0099x; 1.0030x over previous
import math

import jax
import jax.numpy as jnp
from jax import lax
from jax.experimental import pallas as pl
from jax.experimental.pallas import tpu as pltpu
from jax.experimental.pallas import tpu_sc as plsc

NC = 2
NS = 16
NW = NC * NS
LANES = 16
NBUF = 5
SEQ_BLOCKS = 8
POS_BLOCKS = 4


def _make_body(batch, seq, emb):
    seq_per_w = batch // SEQ_BLOCKS
    pos_per_w = seq // POS_BLOCKS
    n_outer = pos_per_w // NBUF
    scale = math.sqrt(emb)
    nvec = emb // LANES

    def body(idx_t_hbm, table_hbm, pos_hbm, out_hbm, ibuf, rows, pos_v, oidx,
             sg0, sg1, sg2, sg3, sg4, sw0, sw1, sw2, sw3, sw4, sem_i, sem_p):
        sem_g = [sg0, sg1, sg2, sg3, sg4]
        sem_w = [sw0, sw1, sw2, sw3, sw4]
        wid = lax.axis_index("s") * NC + lax.axis_index("c")
        sb = lax.rem(wid, SEQ_BLOCKS)
        pb = wid // SEQ_BLOCKS
        seq0 = sb * seq_per_w
        p0 = pb * pos_per_w

        pos_cp = pltpu.async_copy(pos_hbm.at[pb], pos_v, sem_p)

        def fire_idx(p, slot):
            pltpu.async_copy(idx_t_hbm.at[p0 + p, sb], ibuf.at[slot], sem_i)

        def wait_idx():
            pltpu.make_async_copy(idx_t_hbm.at[0, 0], ibuf.at[0],
                                  sem_i).wait()

        def fire_gather(b):
            pltpu.async_copy(table_hbm.at[ibuf.at[b]], rows.at[b], sem_g[b])

        def wait_gather(b):
            pltpu.make_async_copy(out_hbm.at[pl.ds(0, seq_per_w)],
                                  rows.at[0], sem_g[b]).wait()

        hrows = seq_per_w // 2

        def fire_scatter_half(b, h):
            pltpu.async_copy(rows.at[b, pl.ds(h * hrows, hrows)],
                             out_hbm.at[oidx.at[b, h]], sem_w[b])

        def wait_scatter(b):
            pltpu.make_async_copy(rows.at[0], out_hbm.at[pl.ds(0, seq_per_w)],
                                  sem_w[b]).wait()

        lane = lax.iota(jnp.int32, LANES) * seq

        def compute(b, p):
            base = (seq0 * seq) + p0 + p
            pv = [pos_v[p, pl.ds(j * LANES, LANES)] for j in range(nvec)]

            def row_body(r, rc):
                for j in range(nvec):
                    sl = pl.ds(j * LANES, LANES)
                    rows[b, r, sl] = rows[b, r, sl] * scale + pv[j]
                return rc

            for h in range(2):
                for j in range(hrows // LANES):
                    oidx[b, h, pl.ds(j * LANES, LANES)] = lane + (
                        base + (h * hrows + j * LANES) * seq)
                lax.fori_loop(h * hrows, (h + 1) * hrows, row_body, 0)
                fire_scatter_half(b, h)

        fire_idx(0, 0)
        fire_idx(1, 1)
        fire_idx(2, 2)
        wait_idx()
        wait_idx()
        wait_idx()
        fire_gather(0)
        fire_gather(1)
        fire_gather(2)
        fire_idx(3, 3)
        pos_cp.wait()

        def outer(it, c):
            for b in range(NBUF):
                p = it * NBUF + b

                @pl.when(p + 3 < pos_per_w)
                def _():
                    wait_idx()

                    @pl.when(p + 3 >= NBUF)
                    def _():
                        wait_scatter((b + 3) % NBUF)

                    fire_gather((b + 3) % NBUF)

                wait_gather(b)

                @pl.when(p + 4 < pos_per_w)
                def _():
                    fire_idx(p + 4, (b + 4) % NBUF)

                compute(b, p)
            return c

        lax.fori_loop(0, n_outer, outer, 0)
        for b in range(NBUF):
            wait_scatter(b)

    return body


def kernel(to_emb, token_table, pos_table):
    batch, seq = to_emb.shape
    emb = token_table.shape[1]
    seq_per_w = batch // SEQ_BLOCKS
    pos_per_w = seq // POS_BLOCKS
    idx_t = to_emb.T.reshape(seq, SEQ_BLOCKS, seq_per_w)
    pos = pos_table[:seq].reshape(POS_BLOCKS, pos_per_w, emb)

    mesh = plsc.VectorSubcoreMesh(core_axis_name="c", subcore_axis_name="s")
    f = pl.kernel(
        _make_body(batch, seq, emb),
        mesh=mesh,
        out_type=jax.ShapeDtypeStruct((batch * seq, emb), jnp.float32),
        scratch_types=[
            pltpu.VMEM((NBUF, seq_per_w), jnp.int32),
            pltpu.VMEM((NBUF, seq_per_w, emb), jnp.float32),
            pltpu.VMEM((pos_per_w, emb), jnp.float32),
            pltpu.VMEM((NBUF, 2, seq_per_w // 2), jnp.int32),
        ] + [pltpu.SemaphoreType.DMA] * 12,
    )
    return f(idx_t, token_table, pos).reshape(batch, seq, emb)


# speedup vs baseline: 1.0108x; 1.0009x over previous
import math

import jax
import jax.numpy as jnp
from jax import lax
from jax.experimental import pallas as pl
from jax.experimental.pallas import tpu as pltpu
from jax.experimental.pallas import tpu_sc as plsc

NC = 2
NS = 16
NW = NC * NS
LANES = 16
NBUF = 3
PPC = 2
SEQ_BLOCKS = 8
POS_BLOCKS = 4


def _make_body(batch, seq, emb):
    seq_per_w = batch // SEQ_BLOCKS
    pos_per_w = seq // POS_BLOCKS
    n_chunks = pos_per_w // PPC
    rows_per_chunk = PPC * seq_per_w
    scale = math.sqrt(emb)
    nvec = emb // LANES

    def body(idx_t_hbm, table_hbm, pos_hbm, out_hbm, ibuf, rows, pos_v, oidx,
             sg0, sg1, sg2, sw0, sw1, sw2, sem_i, sem_p):
        sem_g = [sg0, sg1, sg2]
        sem_w = [sw0, sw1, sw2]
        wid = lax.axis_index("s") * NC + lax.axis_index("c")
        sb = lax.rem(wid, SEQ_BLOCKS)
        pb = wid // SEQ_BLOCKS
        seq0 = sb * seq_per_w
        p0 = pb * pos_per_w

        pos_cp = pltpu.async_copy(pos_hbm.at[pb], pos_v, sem_p)

        def fire_idx(c, slot):
            pltpu.async_copy(idx_t_hbm.at[pl.ds(p0 + c * PPC, PPC), sb],
                             ibuf.at[slot], sem_i)

        def wait_idx():
            pltpu.make_async_copy(idx_t_hbm.at[pl.ds(0, PPC), 0], ibuf.at[0],
                                  sem_i).wait()

        def fire_gather(c, b):
            pltpu.async_copy(table_hbm.at[ibuf.at[b, 0]],
                             rows.at[b, pl.ds(0, seq_per_w)], sem_g[b])
            pltpu.async_copy(table_hbm.at[ibuf.at[b, 1]],
                             rows.at[b, pl.ds(seq_per_w, seq_per_w)], sem_g[b])

        def wait_gather(b):
            pltpu.make_async_copy(out_hbm.at[pl.ds(0, rows_per_chunk)],
                                  rows.at[0], sem_g[b]).wait()

        def fire_scatter_half(b, h):
            pltpu.async_copy(rows.at[b, pl.ds(h * seq_per_w, seq_per_w)],
                             out_hbm.at[oidx.at[b, h]], sem_w[b])

        def wait_scatter(b):
            pltpu.make_async_copy(rows.at[0],
                                  out_hbm.at[pl.ds(0, rows_per_chunk)],
                                  sem_w[b]).wait()

        lane = lax.iota(jnp.int32, LANES) * seq

        def compute(b, c):
            def row_body(r, rc):
                for j in range(nvec):
                    sl = pl.ds(j * LANES, LANES)
                    rows[b, r, sl] = rows[b, r, sl] * scale + pv[j]
                return rc

            for h in range(PPC):
                p = c * PPC + h
                base = (seq0 * seq) + p0 + p
                pv = [pos_v[p, pl.ds(j * LANES, LANES)] for j in range(nvec)]
                for j in range(nvec):
                    oidx[b, h, pl.ds(j * LANES, LANES)] = lane + (
                        base + j * LANES * seq)
                lax.fori_loop(h * seq_per_w, (h + 1) * seq_per_w, row_body, 0)
                fire_scatter_half(b, h)

        fire_idx(0, 0)
        wait_idx()
        fire_gather(0, 0)
        fire_idx(1, 1)
        pos_cp.wait()

        def outer(it, carry):
            for b in range(NBUF):
                c = it * NBUF + b

                @pl.when(c + 1 < n_chunks)
                def _():
                    wait_idx()

                    @pl.when(c + 1 >= NBUF)
                    def _():
                        wait_scatter((b + 1) % NBUF)

                    fire_gather(c + 1, (b + 1) % NBUF)

                wait_gather(b)

                @pl.when(c + 2 < n_chunks)
                def _():
                    fire_idx(c + 2, (b + 2) % NBUF)

                compute(b, c)
            return carry

        n_outer = (n_chunks - 1) // NBUF
        lax.fori_loop(0, n_outer, outer, 0)
        wait_gather((n_chunks - 1) % NBUF)
        compute((n_chunks - 1) % NBUF, n_chunks - 1)
        for b in range(NBUF):
            wait_scatter(b)

    return body


def kernel(to_emb, token_table, pos_table):
    batch, seq = to_emb.shape
    emb = token_table.shape[1]
    seq_per_w = batch // SEQ_BLOCKS
    pos_per_w = seq // POS_BLOCKS
    idx_t = to_emb.T.reshape(seq, SEQ_BLOCKS, seq_per_w)
    pos = pos_table[:seq].reshape(POS_BLOCKS, pos_per_w, emb)

    mesh = plsc.VectorSubcoreMesh(core_axis_name="c", subcore_axis_name="s")
    f = pl.kernel(
        _make_body(batch, seq, emb),
        mesh=mesh,
        out_type=jax.ShapeDtypeStruct((batch * seq, emb), jnp.float32),
        scratch_types=[
            pltpu.VMEM((NBUF, PPC, seq_per_w), jnp.int32),
            pltpu.VMEM((NBUF, PPC * seq_per_w, emb), jnp.float32),
            pltpu.VMEM((pos_per_w, emb), jnp.float32),
            pltpu.VMEM((NBUF, PPC, seq_per_w), jnp.int32),
        ] + [pltpu.SemaphoreType.DMA] * 8,
    )
    return f(idx_t, token_table, pos).reshape(batch, seq, emb)
